# baseline (device time: 22103 ns/iter reference)
import jax
import jax.numpy as jnp
from jax import lax
from jax.experimental import pallas as pl
from jax.experimental.pallas import tpu as pltpu


def kernel(dy, W):
    m, k = dy.shape
    d, _ = W.shape

    def body(dy_ref, w_ref, out_ref, recv_ref, send_sem, recv_sem):
        my_x = lax.axis_index("x")
        my_y = lax.axis_index("y")
        my_z = lax.axis_index("z")
        peer = (my_x, 1 - my_y, my_z)

        barrier_sem = pltpu.get_barrier_semaphore()
        pl.semaphore_signal(
            barrier_sem, inc=1,
            device_id=peer, device_id_type=pl.DeviceIdType.MESH,
        )
        pl.semaphore_wait(barrier_sem, 1)

        out_ref[...] = lax.dot_general(
            dy_ref[...], w_ref[...],
            dimension_numbers=(((1,), (1,)), ((), ())),
            preferred_element_type=jnp.float32,
        )

        rdma = pltpu.make_async_remote_copy(
            src_ref=out_ref,
            dst_ref=recv_ref,
            send_sem=send_sem,
            recv_sem=recv_sem,
            device_id=peer,
            device_id_type=pl.DeviceIdType.MESH,
        )
        rdma.start()
        rdma.wait()

        out_ref[...] = out_ref[...] + recv_ref[...]

    return pl.pallas_call(
        body,
        out_shape=jax.ShapeDtypeStruct((m, d), jnp.float32),
        in_specs=[
            pl.BlockSpec(memory_space=pltpu.VMEM),
            pl.BlockSpec(memory_space=pltpu.VMEM),
        ],
        out_specs=pl.BlockSpec(memory_space=pltpu.VMEM),
        scratch_shapes=[
            pltpu.VMEM((m, d), jnp.float32),
            pltpu.SemaphoreType.DMA,
            pltpu.SemaphoreType.DMA,
        ],
        compiler_params=pltpu.CompilerParams(collective_id=0),
    )(dy, W)


# device time: 21973 ns/iter; 1.0059x vs baseline; 1.0059x over previous
import jax
import jax.numpy as jnp
from jax import lax
from jax.experimental import pallas as pl
from jax.experimental.pallas import tpu as pltpu

MB = 128
HB = MB // 2


def kernel(dy, W):
    m, k = dy.shape
    d, _ = W.shape

    def body(dy_ref, w_ref, out_ref, yrecv_ref, sems):
        my_x = lax.axis_index("x")
        my_y = lax.axis_index("y")
        my_z = lax.axis_index("z")
        y_peer = (my_x, 1 - my_y, my_z)
        z_peer = (my_x, my_y, 1 - my_z)
        x_peer = (1 - my_x, my_y, my_z)

        b = 2 * my_x + my_z
        b_z = 2 * my_x + (1 - my_z)
        b_x = 2 * (1 - my_x) + my_z
        b_d = 2 * (1 - my_x) + (1 - my_z)

        barrier_sem = pltpu.get_barrier_semaphore()
        for peer in (y_peer, z_peer, x_peer):
            pl.semaphore_signal(
                barrier_sem, inc=1,
                device_id=peer, device_id_type=pl.DeviceIdType.MESH,
            )
        pl.semaphore_wait(barrier_sem, 3)

        out_ref[pl.ds(b * MB, MB), :] = lax.dot_general(
            dy_ref[pl.ds(b * MB, MB), :], w_ref[...],
            dimension_numbers=(((1,), (1,)), ((), ())),
            preferred_element_type=jnp.float32,
        )

        y_rdma = pltpu.make_async_remote_copy(
            src_ref=out_ref.at[pl.ds(b * MB, MB)],
            dst_ref=yrecv_ref,
            send_sem=sems.at[0],
            recv_sem=sems.at[1],
            device_id=y_peer,
            device_id_type=pl.DeviceIdType.MESH,
        )
        y_rdma.start()
        y_rdma.wait()
        out_ref[pl.ds(b * MB, MB), :] = (
            out_ref[pl.ds(b * MB, MB), :] + yrecv_ref[...]
        )

        z_rdma = pltpu.make_async_remote_copy(
            src_ref=out_ref.at[pl.ds(b * MB, MB)],
            dst_ref=out_ref.at[pl.ds(b * MB, MB)],
            send_sem=sems.at[2],
            recv_sem=sems.at[3],
            device_id=z_peer,
            device_id_type=pl.DeviceIdType.MESH,
        )
        x_rdma = pltpu.make_async_remote_copy(
            src_ref=out_ref.at[pl.ds(b * MB, MB)],
            dst_ref=out_ref.at[pl.ds(b * MB, MB)],
            send_sem=sems.at[4],
            recv_sem=sems.at[5],
            device_id=x_peer,
            device_id_type=pl.DeviceIdType.MESH,
        )
        z_rdma.start()
        x_rdma.start()

        z_rdma.wait_recv()
        fwd_x = pltpu.make_async_remote_copy(
            src_ref=out_ref.at[pl.ds(b_z * MB, HB)],
            dst_ref=out_ref.at[pl.ds(b_z * MB, HB)],
            send_sem=sems.at[6],
            recv_sem=sems.at[7],
            device_id=x_peer,
            device_id_type=pl.DeviceIdType.MESH,
        )
        fwd_x.start()

        x_rdma.wait_recv()
        fwd_z = pltpu.make_async_remote_copy(
            src_ref=out_ref.at[pl.ds(b_x * MB + HB, HB)],
            dst_ref=out_ref.at[pl.ds(b_x * MB + HB, HB)],
            send_sem=sems.at[8],
            recv_sem=sems.at[9],
            device_id=z_peer,
            device_id_type=pl.DeviceIdType.MESH,
        )
        fwd_z.start()

        fwd_x.wait_recv()
        fwd_z.wait_recv()

        z_rdma.wait_send()
        x_rdma.wait_send()
        fwd_x.wait_send()
        fwd_z.wait_send()

    return pl.pallas_call(
        body,
        out_shape=jax.ShapeDtypeStruct((m, d), jnp.float32),
        in_specs=[
            pl.BlockSpec(memory_space=pltpu.VMEM),
            pl.BlockSpec(memory_space=pltpu.VMEM),
        ],
        out_specs=pl.BlockSpec(memory_space=pltpu.VMEM),
        scratch_shapes=[
            pltpu.VMEM((MB, d), jnp.float32),
            pltpu.SemaphoreType.DMA((10,)),
        ],
        compiler_params=pltpu.CompilerParams(collective_id=0),
    )(dy, W)


# device time: 19247 ns/iter; 1.1484x vs baseline; 1.1416x over previous
import jax
import jax.numpy as jnp
from jax import lax
from jax.experimental import pallas as pl
from jax.experimental.pallas import tpu as pltpu

MB = 128
NC = 4
CS = MB // NC
HC = CS // 2


def kernel(dy, W):
    m, k = dy.shape
    d, _ = W.shape

    def body(dy_ref, w_ref, out_ref, yrecv_ref, ysem, zsem, xsem, fxsem, fzsem):
        my_x = lax.axis_index("x")
        my_y = lax.axis_index("y")
        my_z = lax.axis_index("z")
        y_peer = (my_x, 1 - my_y, my_z)
        z_peer = (my_x, my_y, 1 - my_z)
        x_peer = (1 - my_x, my_y, my_z)

        b = 2 * my_x + my_z
        b_z = 2 * my_x + (1 - my_z)
        b_x = 2 * (1 - my_x) + my_z

        barrier_sem = pltpu.get_barrier_semaphore()
        for peer in (y_peer, z_peer, x_peer):
            pl.semaphore_signal(
                barrier_sem, inc=1,
                device_id=peer, device_id_type=pl.DeviceIdType.MESH,
            )
        pl.semaphore_wait(barrier_sem, 3)

        def y_rdma(c):
            return pltpu.make_async_remote_copy(
                src_ref=out_ref.at[pl.ds(b * MB + c * CS, CS)],
                dst_ref=yrecv_ref.at[pl.ds(c * CS, CS)],
                send_sem=ysem.at[2 * c],
                recv_sem=ysem.at[2 * c + 1],
                device_id=y_peer,
                device_id_type=pl.DeviceIdType.MESH,
            )

        def direct_rdma(c, sems, peer):
            return pltpu.make_async_remote_copy(
                src_ref=out_ref.at[pl.ds(b * MB + c * CS, CS)],
                dst_ref=out_ref.at[pl.ds(b * MB + c * CS, CS)],
                send_sem=sems.at[2 * c],
                recv_sem=sems.at[2 * c + 1],
                device_id=peer,
                device_id_type=pl.DeviceIdType.MESH,
            )

        def fwd_x_rdma(c):
            return pltpu.make_async_remote_copy(
                src_ref=out_ref.at[pl.ds(b_z * MB + c * CS, HC)],
                dst_ref=out_ref.at[pl.ds(b_z * MB + c * CS, HC)],
                send_sem=fxsem.at[2 * c],
                recv_sem=fxsem.at[2 * c + 1],
                device_id=x_peer,
                device_id_type=pl.DeviceIdType.MESH,
            )

        def fwd_z_rdma(c):
            return pltpu.make_async_remote_copy(
                src_ref=out_ref.at[pl.ds(b_x * MB + c * CS + HC, HC)],
                dst_ref=out_ref.at[pl.ds(b_x * MB + c * CS + HC, HC)],
                send_sem=fzsem.at[2 * c],
                recv_sem=fzsem.at[2 * c + 1],
                device_id=z_peer,
                device_id_type=pl.DeviceIdType.MESH,
            )

        for c in range(NC):
            out_ref[pl.ds(b * MB + c * CS, CS), :] = lax.dot_general(
                dy_ref[pl.ds(b * MB + c * CS, CS), :], w_ref[...],
                dimension_numbers=(((1,), (1,)), ((), ())),
                preferred_element_type=jnp.float32,
            )
            y_rdma(c).start()

        zs, xs = [], []
        for c in range(NC):
            y_rdma(c).wait_recv()
            out_ref[pl.ds(b * MB + c * CS, CS), :] = (
                out_ref[pl.ds(b * MB + c * CS, CS), :]
                + yrecv_ref[pl.ds(c * CS, CS), :]
            )
            z = direct_rdma(c, zsem, z_peer)
            x = direct_rdma(c, xsem, x_peer)
            z.start()
            x.start()
            zs.append(z)
            xs.append(x)

        fxs, fzs = [], []
        for c in range(NC):
            zs[c].wait_recv()
            fx = fwd_x_rdma(c)
            fx.start()
            fxs.append(fx)
            xs[c].wait_recv()
            fz = fwd_z_rdma(c)
            fz.start()
            fzs.append(fz)

        for c in range(NC):
            fxs[c].wait_recv()
            fzs[c].wait_recv()
        for c in range(NC):
            y_rdma(c).wait_send()
            zs[c].wait_send()
            xs[c].wait_send()
            fxs[c].wait_send()
            fzs[c].wait_send()

    return pl.pallas_call(
        body,
        out_shape=jax.ShapeDtypeStruct((m, d), jnp.float32),
        in_specs=[
            pl.BlockSpec(memory_space=pltpu.VMEM),
            pl.BlockSpec(memory_space=pltpu.VMEM),
        ],
        out_specs=pl.BlockSpec(memory_space=pltpu.VMEM),
        scratch_shapes=[
            pltpu.VMEM((MB, d), jnp.float32),
            pltpu.SemaphoreType.DMA((2 * NC,)),
            pltpu.SemaphoreType.DMA((2 * NC,)),
            pltpu.SemaphoreType.DMA((2 * NC,)),
            pltpu.SemaphoreType.DMA((2 * NC,)),
            pltpu.SemaphoreType.DMA((2 * NC,)),
        ],
        compiler_params=pltpu.CompilerParams(collective_id=0),
    )(dy, W)


# device time: 18218 ns/iter; 1.2133x vs baseline; 1.0565x over previous
import jax
import jax.numpy as jnp
from jax import lax
from jax.experimental import pallas as pl
from jax.experimental.pallas import tpu as pltpu

MB = 128
NC = 4
CS = MB // NC
HC = CS // 2


def kernel(dy, W):
    m, k = dy.shape
    d, _ = W.shape

    def body(dy_ref, w_ref, out_ref, ypart_ref, yprecv_ref, comm_ref,
             ysem, zsem, xsem, fxsem, fzsem):
        my_x = lax.axis_index("x")
        my_y = lax.axis_index("y")
        my_z = lax.axis_index("z")
        y_peer = (my_x, 1 - my_y, my_z)
        z_peer = (my_x, my_y, 1 - my_z)
        x_peer = (1 - my_x, my_y, my_z)

        b = 2 * my_x + my_z
        b_z = 2 * my_x + (1 - my_z)
        b_x = 2 * (1 - my_x) + my_z
        b_d = 2 * (1 - my_x) + (1 - my_z)

        barrier_sem = pltpu.get_barrier_semaphore()
        for peer in (y_peer, z_peer, x_peer):
            pl.semaphore_signal(
                barrier_sem, inc=1,
                device_id=peer, device_id_type=pl.DeviceIdType.MESH,
            )
        pl.semaphore_wait(barrier_sem, 3)

        def y_rdma(c):
            return pltpu.make_async_remote_copy(
                src_ref=ypart_ref.at[pl.ds(c * CS, CS)],
                dst_ref=yprecv_ref.at[pl.ds(c * CS, CS)],
                send_sem=ysem.at[2 * c],
                recv_sem=ysem.at[2 * c + 1],
                device_id=y_peer,
                device_id_type=pl.DeviceIdType.MESH,
            )

        def direct_rdma(c, sems, peer):
            return pltpu.make_async_remote_copy(
                src_ref=comm_ref.at[pl.ds(b * MB + c * CS, CS)],
                dst_ref=comm_ref.at[pl.ds(b * MB + c * CS, CS)],
                send_sem=sems.at[2 * c],
                recv_sem=sems.at[2 * c + 1],
                device_id=peer,
                device_id_type=pl.DeviceIdType.MESH,
            )

        def fwd_x_rdma(c):
            return pltpu.make_async_remote_copy(
                src_ref=comm_ref.at[pl.ds(b_z * MB + c * CS, HC)],
                dst_ref=comm_ref.at[pl.ds(b_z * MB + c * CS, HC)],
                send_sem=fxsem.at[2 * c],
                recv_sem=fxsem.at[2 * c + 1],
                device_id=x_peer,
                device_id_type=pl.DeviceIdType.MESH,
            )

        def fwd_z_rdma(c):
            return pltpu.make_async_remote_copy(
                src_ref=comm_ref.at[pl.ds(b_x * MB + c * CS + HC, HC)],
                dst_ref=comm_ref.at[pl.ds(b_x * MB + c * CS + HC, HC)],
                send_sem=fzsem.at[2 * c],
                recv_sem=fzsem.at[2 * c + 1],
                device_id=z_peer,
                device_id_type=pl.DeviceIdType.MESH,
            )

        for c in range(NC):
            rows = pl.ds(b * MB + c * CS, CS)
            out_ref[rows, :] = lax.dot_general(
                dy_ref[rows, :], w_ref[...],
                dimension_numbers=(((1,), (1,)), ((), ())),
                preferred_element_type=jnp.float32,
            )
            ypart_ref[pl.ds(c * CS, CS), :] = out_ref[rows, :].astype(
                jnp.bfloat16
            )
            y_rdma(c).start()

        zs, xs = [], []
        for c in range(NC):
            rows = pl.ds(b * MB + c * CS, CS)
            y_rdma(c).wait_recv()
            acc = out_ref[rows, :] + yprecv_ref[pl.ds(c * CS, CS), :].astype(
                jnp.float32
            )
            out_ref[rows, :] = acc
            comm_ref[rows, :] = acc.astype(jnp.bfloat16)
            z = direct_rdma(c, zsem, z_peer)
            x = direct_rdma(c, xsem, x_peer)
            z.start()
            x.start()
            zs.append(z)
            xs.append(x)

        fxs, fzs = [], []
        for c in range(NC):
            zs[c].wait_recv()
            fx = fwd_x_rdma(c)
            fx.start()
            fxs.append(fx)
            zrows = pl.ds(b_z * MB + c * CS, CS)
            out_ref[zrows, :] = comm_ref[zrows, :].astype(jnp.float32)
            xs[c].wait_recv()
            fz = fwd_z_rdma(c)
            fz.start()
            fzs.append(fz)
            xrows = pl.ds(b_x * MB + c * CS, CS)
            out_ref[xrows, :] = comm_ref[xrows, :].astype(jnp.float32)

        for c in range(NC):
            fxs[c].wait_recv()
            fzs[c].wait_recv()
            drows = pl.ds(b_d * MB + c * CS, CS)
            out_ref[drows, :] = comm_ref[drows, :].astype(jnp.float32)
        for c in range(NC):
            y_rdma(c).wait_send()
            zs[c].wait_send()
            xs[c].wait_send()
            fxs[c].wait_send()
            fzs[c].wait_send()

    return pl.pallas_call(
        body,
        out_shape=jax.ShapeDtypeStruct((m, d), jnp.float32),
        in_specs=[
            pl.BlockSpec(memory_space=pltpu.VMEM),
            pl.BlockSpec(memory_space=pltpu.VMEM),
        ],
        out_specs=pl.BlockSpec(memory_space=pltpu.VMEM),
        scratch_shapes=[
            pltpu.VMEM((MB, d), jnp.bfloat16),
            pltpu.VMEM((MB, d), jnp.bfloat16),
            pltpu.VMEM((m, d), jnp.bfloat16),
            pltpu.SemaphoreType.DMA((2 * NC,)),
            pltpu.SemaphoreType.DMA((2 * NC,)),
            pltpu.SemaphoreType.DMA((2 * NC,)),
            pltpu.SemaphoreType.DMA((2 * NC,)),
            pltpu.SemaphoreType.DMA((2 * NC,)),
        ],
        compiler_params=pltpu.CompilerParams(collective_id=0),
    )(dy, W)
